# baseline (speedup 1.0000x reference)
import jax
import jax.numpy as jnp
from jax import lax
from jax.experimental import pallas as pl
from jax.experimental.pallas import tpu as pltpu
from jax.experimental.pallas import tpu_sc as plsc

R = 1024
C = 100000
B = 128
NC = 2
NS = 16
NW = NC * NS
ROWS_PER_W = R // NW
WIN = 128
NBUF = 6


def _gather_body(dataT, idx_hbm, out_hbm, idx_v, out_v, stage_sh,
                 buf0, buf1, buf2, buf3, buf4, buf5,
                 sem0, sem1, sem2, sem3, sem4, sem5,
                 csem0, csem1, csem2, csem3, csem4, csem5):
    c = lax.axis_index("c")
    s = lax.axis_index("s")
    wid = s * NC + c
    row0 = wid * ROWS_PER_W
    rblk = pl.multiple_of((row0 // WIN) * WIN, WIN)
    off0 = row0 - rblk

    bufs = (buf0, buf1, buf2, buf3, buf4, buf5)
    sems = (sem0, sem1, sem2, sem3, sem4, sem5)
    csems = (csem0, csem1, csem2, csem3, csem4, csem5)

    pltpu.sync_copy(idx_hbm.at[pl.ds(row0, ROWS_PER_W)], idx_v)

    def fire(i, b):
        pltpu.async_copy(
            dataT.at[idx_v.at[i], pl.ds(rblk, WIN)], bufs[b], sems[b]
        )

    def drain(i, b):
        pltpu.make_async_copy(
            dataT.at[idx_v.at[i], pl.ds(rblk, WIN)], bufs[b], sems[b]
        ).wait()

    def col_copy(i, b):
        pltpu.async_copy(bufs[b].at[:, off0 + i], stage_sh.at[s, i], csems[b])

    def col_wait(b):
        pltpu.make_async_copy(
            bufs[b].at[:, 0], stage_sh.at[s, 0], csems[b]
        ).wait()

    for b in range(NBUF):
        fire(b, b)

    def step(i, carry):
        b = lax.rem(i, NBUF)

        nxt = i + NBUF - 1

        @pl.when(jnp.logical_and(i > 0, nxt < ROWS_PER_W))
        def _():
            bn = lax.rem(nxt, NBUF)
            col_wait_dyn(bn)
            fire_dyn(nxt, bn)

        drain_dyn(i, b)
        col_copy_dyn(i, b)
        return carry

    def _dispatch(fn):
        def run(*args):
            b = args[-1]
            for bb in range(NBUF):
                @pl.when(b == bb)
                def _():
                    fn(*args[:-1], bb)
        return run

    fire_dyn = _dispatch(fire)
    drain_dyn = _dispatch(drain)
    col_copy_dyn = _dispatch(col_copy)
    col_wait_dyn = _dispatch(lambda b: col_wait(b))

    lax.fori_loop(0, ROWS_PER_W, step, 0)

    for b in range(NBUF):
        col_wait(b)

    pltpu.sync_copy(stage_sh.at[s], out_v)
    pltpu.sync_copy(out_v, out_hbm.at[pl.ds(row0, ROWS_PER_W)])


@jax.jit
def _gather(dataT, idx):
    mesh = plsc.VectorSubcoreMesh(core_axis_name="c", subcore_axis_name="s")
    return pl.kernel(
        _gather_body,
        mesh=mesh,
        out_type=jax.ShapeDtypeStruct((R, B), jnp.float32),
        scratch_types=[
            pltpu.VMEM((ROWS_PER_W, B), jnp.int32),
            pltpu.VMEM((ROWS_PER_W, B), jnp.float32),
            pltpu.VMEM_SHARED((NS, ROWS_PER_W, B), jnp.float32),
        ]
        + [pltpu.VMEM((B, WIN), jnp.float32)] * NBUF
        + [pltpu.SemaphoreType.DMA] * (2 * NBUF),
    )(dataT, idx)


def kernel(data, idx):
    return _gather(data.T, idx)


# speedup vs baseline: 1.0641x; 1.0641x over previous
import jax
import jax.numpy as jnp
from jax import lax
from jax.experimental import pallas as pl
from jax.experimental.pallas import tpu as pltpu
from jax.experimental.pallas import tpu_sc as plsc

R = 1024
C = 100000
B = 128
NC = 2
NS = 16
NW = NC * NS
ROWS_PER_W = R // NW
WIN = 128
NBUF = 4


def _gather_body(dataT, idx_hbm, out_hbm, idx_v, out_v, stage_sh,
                 buf0, buf1, buf2, buf3, sem0, sem1, sem2, sem3):
    c = lax.axis_index("c")
    s = lax.axis_index("s")
    wid = s * NC + c
    row0 = wid * ROWS_PER_W
    rblk = pl.multiple_of((row0 // WIN) * WIN, WIN)
    off0 = row0 - rblk

    bufs = (buf0, buf1, buf2, buf3)
    sems = (sem0, sem1, sem2, sem3)

    pltpu.sync_copy(idx_hbm.at[pl.ds(row0, ROWS_PER_W)], idx_v)

    def fire(i, b):
        pltpu.async_copy(
            dataT.at[idx_v.at[i], pl.ds(rblk, WIN)], bufs[b], sems[b]
        )

    def drain(i, b):
        pltpu.make_async_copy(
            dataT.at[idx_v.at[i], pl.ds(rblk, WIN)], bufs[b], sems[b]
        ).wait()

    for b in range(NBUF):
        fire(b, b)

    def group(g, carry):
        for b in range(NBUF):
            i = g * NBUF + b
            drain(i, b)
            pltpu.sync_copy(bufs[b].at[:, off0 + i], stage_sh.at[s, i])

            @pl.when(i + NBUF < ROWS_PER_W)
            def _():
                fire(i + NBUF, b)
        return carry

    lax.fori_loop(0, ROWS_PER_W // NBUF, group, 0)

    pltpu.sync_copy(stage_sh.at[s], out_v)
    pltpu.sync_copy(out_v, out_hbm.at[pl.ds(row0, ROWS_PER_W)])


@jax.jit
def _gather(dataT, idx):
    mesh = plsc.VectorSubcoreMesh(core_axis_name="c", subcore_axis_name="s")
    return pl.kernel(
        _gather_body,
        mesh=mesh,
        out_type=jax.ShapeDtypeStruct((R, B), jnp.float32),
        scratch_types=[
            pltpu.VMEM((ROWS_PER_W, B), jnp.int32),
            pltpu.VMEM((ROWS_PER_W, B), jnp.float32),
            pltpu.VMEM_SHARED((NS, ROWS_PER_W, B), jnp.float32),
            pltpu.VMEM((B, WIN), jnp.float32),
            pltpu.VMEM((B, WIN), jnp.float32),
            pltpu.VMEM((B, WIN), jnp.float32),
            pltpu.VMEM((B, WIN), jnp.float32),
            pltpu.SemaphoreType.DMA,
            pltpu.SemaphoreType.DMA,
            pltpu.SemaphoreType.DMA,
            pltpu.SemaphoreType.DMA,
        ],
    )(dataT, idx)


def kernel(data, idx):
    return _gather(data.T, idx)
